# Initial kernel scaffold; baseline (speedup 1.0000x reference)
import jax
import jax.numpy as jnp
from jax.experimental import pallas as pl


def kernel(x, w_gate_up, w_down, retain_gate, retain_up, retain_down, forget_gate, forget_up, forget_down, scales, token_experiment_ids):
    raise NotImplementedError("write your pallas kernel here")


# speedup vs baseline: 1.8000x; 1.8000x over previous
import jax
import jax.numpy as jnp
from jax.experimental import pallas as pl
from jax.experimental.pallas import tpu as pltpu

_F32 = jnp.float32
_BF16 = jnp.bfloat16


def _silu(g):
    return g * (1.0 / (1.0 + jnp.exp(-g)))


def _base_body(x_ref, wg_ref, wu_ref, wd_ref, o_ref):
    f = pl.program_id(0)
    g = jnp.dot(x_ref[...], wg_ref[...], preferred_element_type=_F32)
    u = jnp.dot(x_ref[...], wu_ref[...], preferred_element_type=_F32)
    h = (_silu(g) * u).astype(_BF16)
    contrib = jnp.dot(h, wd_ref[...], preferred_element_type=_F32)

    @pl.when(f == 0)
    def _():
        o_ref[...] = contrib

    @pl.when(f > 0)
    def _():
        o_ref[...] += contrib


def _adapter_body(ids_ref, sv_ref, x_ref, b_ref, wg_ref, wu_ref, wd_ref, o_ref):
    e = pl.program_id(0)
    g = jnp.dot(x_ref[...], wg_ref[0], preferred_element_type=_F32)
    u = jnp.dot(x_ref[...], wu_ref[0], preferred_element_type=_F32)
    inter = _silu(g) * u * sv_ref[0]
    mask = ids_ref[...] == e
    inter = jnp.where(mask, inter, 0.0).astype(_BF16)
    contrib = jnp.dot(inter, wd_ref[0], preferred_element_type=_F32)

    @pl.when(e == 0)
    def _():
        o_ref[...] = b_ref[...] + contrib

    @pl.when(e > 0)
    def _():
        o_ref[...] += contrib


def kernel(x, w_gate_up, w_down, retain_gate, retain_up, retain_down,
           forget_gate, forget_up, forget_down, scales, token_experiment_ids):
    T, H = x.shape
    FF = w_down.shape[0]
    A, NR, _ = retain_gate.shape
    NFG = forget_gate.shape[1]
    N2 = NR + NFG

    xb = x.astype(_BF16)
    wg = w_gate_up[:, :FF].astype(_BF16)
    wu = w_gate_up[:, FF:].astype(_BF16)
    wd = w_down.astype(_BF16)

    NFB = 8
    BF = FF // NFB

    base = pl.pallas_call(
        _base_body,
        grid=(NFB,),
        in_specs=[
            pl.BlockSpec((T, H), lambda f: (0, 0)),
            pl.BlockSpec((H, BF), lambda f: (0, f)),
            pl.BlockSpec((H, BF), lambda f: (0, f)),
            pl.BlockSpec((BF, H), lambda f: (f, 0)),
        ],
        out_specs=pl.BlockSpec((T, H), lambda f: (0, 0)),
        out_shape=jax.ShapeDtypeStruct((T, H), _F32),
        compiler_params=pltpu.CompilerParams(
            dimension_semantics=("arbitrary",)),
    )(xb, wg, wu, wd)

    Wg_a = jnp.concatenate([retain_gate, forget_gate], axis=1).transpose(0, 2, 1).astype(_BF16)
    Wu_a = jnp.concatenate([retain_up, forget_up], axis=1).transpose(0, 2, 1).astype(_BF16)
    Wd_a = jnp.concatenate([retain_down.transpose(0, 2, 1),
                            forget_down.transpose(0, 2, 1)], axis=1).astype(_BF16)
    scale_vec = jnp.concatenate([jnp.repeat(scales[:, 0:1], NR, axis=1),
                                 jnp.repeat(scales[:, 1:2], NFG, axis=1)], axis=1)
    scale_vec = scale_vec.reshape(A, 1, N2)
    ids_col = token_experiment_ids.astype(jnp.int32).reshape(T, 1)

    out = pl.pallas_call(
        _adapter_body,
        grid=(A,),
        in_specs=[
            pl.BlockSpec((T, 1), lambda e: (0, 0)),
            pl.BlockSpec((1, 1, N2), lambda e: (e, 0, 0)),
            pl.BlockSpec((T, H), lambda e: (0, 0)),
            pl.BlockSpec((T, H), lambda e: (0, 0)),
            pl.BlockSpec((1, H, N2), lambda e: (e, 0, 0)),
            pl.BlockSpec((1, H, N2), lambda e: (e, 0, 0)),
            pl.BlockSpec((1, N2, H), lambda e: (e, 0, 0)),
        ],
        out_specs=pl.BlockSpec((T, H), lambda e: (0, 0)),
        out_shape=jax.ShapeDtypeStruct((T, H), _F32),
        compiler_params=pltpu.CompilerParams(
            dimension_semantics=("arbitrary",)),
    )(ids_col, scale_vec, xb, base, Wg_a, Wu_a, Wd_a)

    return out
